# Initial kernel scaffold; baseline (speedup 1.0000x reference)
import jax
import jax.numpy as jnp
from jax.experimental import pallas as pl


def kernel(node_embs, memory, W_ih, W_hh, b_ih, b_hh, W_dec, b_dec, ids):
    raise NotImplementedError("write your pallas kernel here")


# speedup vs baseline: 1.0957x; 1.0957x over previous
import jax
import jax.numpy as jnp
from jax import lax
from jax.experimental import pallas as pl

N_NODES = 1_000_000
HIDDEN_D = 32
INPUT_D = 32
N_LAYERS = 2
BATCH = 16384
OUT_D = 32


def _rnn_body(x_ref, h_ref, wih_ref, whh_ref, bih_ref, bhh_ref,
              wdec_ref, bdec_ref, out_ref, st_ref):
  f32 = jnp.float32
  x = x_ref[...]
  h0 = h_ref[:, 0, :]
  h1 = h_ref[:, 1, :]
  h0n = jnp.tanh(
      jnp.dot(x, wih_ref[0], preferred_element_type=f32) + bih_ref[0]
      + jnp.dot(h0, whh_ref[0], preferred_element_type=f32) + bhh_ref[0])
  h1n = jnp.tanh(
      jnp.dot(h0n, wih_ref[1], preferred_element_type=f32) + bih_ref[1]
      + jnp.dot(h1, whh_ref[1], preferred_element_type=f32) + bhh_ref[1])
  out_ref[...] = (jnp.dot(h1n, wdec_ref[...], preferred_element_type=f32)
                  + bdec_ref[...])
  st_ref[:, 0, :] = h0n
  st_ref[:, 1, :] = h1n


_rnn_call = pl.pallas_call(
    _rnn_body,
    out_shape=(
        jax.ShapeDtypeStruct((BATCH, OUT_D), jnp.float32),
        jax.ShapeDtypeStruct((BATCH, N_LAYERS, HIDDEN_D), jnp.float32),
    ),
)


def kernel(node_embs, memory, W_ih, W_hh, b_ih, b_hh, W_dec, b_dec, ids):
  x = jnp.take(node_embs, ids, axis=0)
  h = jnp.take(memory, ids, axis=0)
  out, stacked = _rnn_call(x, h, W_ih, W_hh, b_ih, b_hh, W_dec, b_dec)
  new_memory = memory.at[ids].set(stacked)
  return out, new_memory
